# Initial kernel scaffold; baseline (speedup 1.0000x reference)
import jax
import jax.numpy as jnp
from jax.experimental import pallas as pl


def kernel(idx, targets, W):
    raise NotImplementedError("write your pallas kernel here")


# speedup vs baseline: 1.9601x; 1.9601x over previous
import functools

import jax
import jax.numpy as jnp
from jax import lax
from jax.experimental import pallas as pl
from jax.experimental.pallas import tpu as pltpu
from jax.experimental.pallas import tpu_sc as plsc

_VOCAB = 4096
_N = 8 * 2048
_NC, _NS, _L = 2, 16, 16
_NW = _NC * _NS
_ROWS_PER_W = _N // _NW
_K = 16
_NCHUNK = _ROWS_PER_W // _K


def _lse_body(w_ref, out_ref):
    x = w_ref[...]
    m = jnp.max(x, axis=1, keepdims=True)
    s = jnp.sum(jnp.exp(x - m), axis=1, keepdims=True)
    out_ref[...] = m + jnp.log(s)


_R = 256


def _compute_lse(W):
    out = pl.pallas_call(
        _lse_body,
        grid=(_VOCAB // _R,),
        in_specs=[pl.BlockSpec((_R, _VOCAB), lambda i: (i, 0))],
        out_specs=pl.BlockSpec((_R, 1), lambda i: (i, 0)),
        out_shape=jax.ShapeDtypeStruct((_VOCAB, 1), jnp.float32),
    )(W)
    return out.reshape(_VOCAB)


@functools.partial(
    pl.kernel,
    mesh=plsc.VectorSubcoreMesh(core_axis_name="c", subcore_axis_name="s"),
    out_type=[
        jax.ShapeDtypeStruct((_N, _VOCAB), jnp.float32),
        jax.ShapeDtypeStruct((_NW, _L), jnp.float32),
    ],
    scratch_types=[
        pltpu.VMEM((_ROWS_PER_W,), jnp.int32),
        pltpu.VMEM((_ROWS_PER_W,), jnp.int32),
        pltpu.VMEM((_ROWS_PER_W,), jnp.float32),
        pltpu.VMEM((_ROWS_PER_W,), jnp.float32),
        pltpu.VMEM((_K, _VOCAB), jnp.float32),
        pltpu.VMEM((_L,), jnp.float32),
        pltpu.SemaphoreType.DMA,
        pltpu.SemaphoreType.DMA,
    ],
)
def _sc_gather(W_hbm, Wflat_hbm, idx_hbm, tgt_hbm, lse_hbm, out_hbm, part_hbm,
               idx_v, flat_v, tv_v, lv_v, rows_v, acc_v, sem, sem2):
    wid = lax.axis_index("s") * _NC + lax.axis_index("c")
    base = wid * _ROWS_PER_W
    pltpu.sync_copy(idx_hbm.at[pl.ds(base, _ROWS_PER_W)], idx_v)
    pltpu.sync_copy(tgt_hbm.at[pl.ds(base, _ROWS_PER_W)], flat_v)

    def fbody(j, _):
        off = j * _L
        flat_v[pl.ds(off, _L)] = (idx_v[pl.ds(off, _L)] * _VOCAB
                                  + flat_v[pl.ds(off, _L)])
        return 0

    lax.fori_loop(0, _ROWS_PER_W // _L, fbody, 0)

    cp_t = pltpu.async_copy(Wflat_hbm.at[flat_v], tv_v, sem2)
    cp_l = pltpu.async_copy(lse_hbm.at[idx_v], lv_v, sem)
    cp_t.wait()
    cp_l.wait()

    acc_v[...] = jnp.zeros((_L,), jnp.float32)

    def body(g, _):
        off = g * _K
        pltpu.async_copy(W_hbm.at[idx_v.at[pl.ds(off, _K)]], rows_v, sem).wait()
        pltpu.sync_copy(rows_v, out_hbm.at[pl.ds(base + off, _K)])
        acc_v[...] = acc_v[...] + (lv_v[pl.ds(off, _L)] - tv_v[pl.ds(off, _L)])
        return 0

    lax.fori_loop(0, _NCHUNK, body, 0)
    pltpu.sync_copy(acc_v, part_hbm.at[wid])


def kernel(idx, targets, W):
    idx_flat = idx.reshape(_N).astype(jnp.int32)
    tgt_flat = targets.reshape(_N).astype(jnp.int32)
    lse = _compute_lse(W)
    logits_flat, partials = _sc_gather(
        W, W.reshape(_VOCAB * _VOCAB), idx_flat, tgt_flat, lse)
    loss = jnp.sum(partials) / _N
    return (logits_flat, loss)


# speedup vs baseline: 2.0339x; 1.0377x over previous
import functools

import jax
import jax.numpy as jnp
from jax import lax
from jax.experimental import pallas as pl
from jax.experimental.pallas import tpu as pltpu
from jax.experimental.pallas import tpu_sc as plsc

_VOCAB = 4096
_N = 8 * 2048
_NC, _NS, _L = 2, 16, 16
_NW = _NC * _NS
_ROWS_PER_W = _N // _NW
_K = 8
_NCHUNK = _ROWS_PER_W // _K
_NPAIR = _NCHUNK // 2
_TPAD = _ROWS_PER_W + _L


def _lse_body(w_ref, out_ref):
    x = w_ref[...]
    m = jnp.max(x, axis=1, keepdims=True)
    s = jnp.sum(jnp.exp(x - m), axis=1, keepdims=True)
    out_ref[...] = m + jnp.log(s)


_R = 256


def _compute_lse(W):
    out = pl.pallas_call(
        _lse_body,
        grid=(_VOCAB // _R,),
        in_specs=[pl.BlockSpec((_R, _VOCAB), lambda i: (i, 0))],
        out_specs=pl.BlockSpec((_R, 1), lambda i: (i, 0)),
        out_shape=jax.ShapeDtypeStruct((_VOCAB, 1), jnp.float32),
    )(W)
    return out.reshape(_VOCAB)


@functools.partial(
    pl.kernel,
    mesh=plsc.VectorSubcoreMesh(core_axis_name="c", subcore_axis_name="s"),
    out_type=[
        jax.ShapeDtypeStruct((_N, _VOCAB), jnp.float32),
        jax.ShapeDtypeStruct((_NW, _L), jnp.float32),
    ],
    scratch_types=[
        pltpu.VMEM((_ROWS_PER_W,), jnp.int32),
        pltpu.VMEM((_ROWS_PER_W,), jnp.int32),
        pltpu.VMEM((_ROWS_PER_W,), jnp.float32),
        pltpu.VMEM((_ROWS_PER_W,), jnp.float32),
        pltpu.VMEM((_K, _VOCAB), jnp.float32),
        pltpu.VMEM((_K, _VOCAB), jnp.float32),
        pltpu.VMEM((_L,), jnp.float32),
        pltpu.SemaphoreType.DMA,
        pltpu.SemaphoreType.DMA,
        pltpu.SemaphoreType.DMA,
        pltpu.SemaphoreType.DMA,
        pltpu.SemaphoreType.DMA,
    ],
)
def _sc_gather(W_hbm, Wflat_hbm, idx_hbm, tgt_hbm, lse_hbm, out_hbm, part_hbm,
               idx_v, flat_v, tv_v, lv_v, rows_a, rows_b, acc_v,
               gsem_a, gsem_b, wsem_a, wsem_b, lsem):
    wid = lax.axis_index("s") * _NC + lax.axis_index("c")
    base = wid * _ROWS_PER_W
    pltpu.sync_copy(idx_hbm.at[pl.ds(base, _ROWS_PER_W)], idx_v)
    pltpu.sync_copy(tgt_hbm.at[pl.ds(base, _ROWS_PER_W)], flat_v)

    def fbody(j, _):
        off = j * _L
        flat_v[pl.ds(off, _L)] = (idx_v[pl.ds(off, _L)] * _VOCAB
                                  + flat_v[pl.ds(off, _L)])
        return 0

    lax.fori_loop(0, _ROWS_PER_W // _L, fbody, 0)

    cp_t = pltpu.async_copy(Wflat_hbm.at[flat_v], tv_v, lsem)
    cp_l = pltpu.async_copy(lse_hbm.at[idx_v], lv_v, lsem)

    lanes = lax.iota(jnp.int32, _L)

    def _gather(off, buf, sem):
        return pltpu.async_copy(W_hbm.at[idx_v.at[pl.ds(off, _K)]], buf, sem)

    def _gather_wait(off, buf, sem):
        pltpu.make_async_copy(W_hbm.at[idx_v.at[pl.ds(off, _K)]], buf,
                              sem).wait()

    def _wb(off, buf, sem):
        return pltpu.async_copy(buf, out_hbm.at[pl.ds(base + off, _K)], sem)

    def _wb_wait(off, buf, sem):
        pltpu.make_async_copy(buf, out_hbm.at[pl.ds(base + off, _K)],
                              sem).wait()

    _gather(0, rows_a, gsem_a)

    def body(m, _):
        e_off = (2 * m) * _K
        o_off = e_off + _K
        _gather(o_off, rows_b, gsem_b)
        _gather_wait(e_off, rows_a, gsem_a)
        _wb(e_off, rows_a, wsem_a)
        _gather_wait(o_off, rows_b, gsem_b)
        _wb(o_off, rows_b, wsem_b)
        _wb_wait(e_off, rows_a, wsem_a)

        @pl.when(m < _NPAIR - 1)
        def _():
            _gather(e_off + 2 * _K, rows_a, gsem_a)

        _wb_wait(o_off, rows_b, wsem_b)
        return 0

    lax.fori_loop(0, _NPAIR, body, 0)

    cp_t.wait()
    cp_l.wait()

    def lbody(j, accv):
        off = j * _L
        return accv + (lv_v[pl.ds(off, _L)] - tv_v[pl.ds(off, _L)])

    accv = lax.fori_loop(0, _ROWS_PER_W // _L, lbody,
                         jnp.zeros((_L,), jnp.float32))
    acc_v[...] = accv
    pltpu.sync_copy(acc_v, part_hbm.at[wid])


def kernel(idx, targets, W):
    idx_flat = idx.reshape(_N).astype(jnp.int32)
    tgt_flat = targets.reshape(_N).astype(jnp.int32)
    lse = _compute_lse(W)
    logits_flat, partials = _sc_gather(
        W, W.reshape(_VOCAB * _VOCAB), idx_flat, tgt_flat, lse)
    loss = jnp.sum(partials) / _N
    return (logits_flat, loss)


# speedup vs baseline: 2.3415x; 1.1512x over previous
import functools

import jax
import jax.numpy as jnp
from jax import lax
from jax.experimental import pallas as pl
from jax.experimental.pallas import tpu as pltpu
from jax.experimental.pallas import tpu_sc as plsc

_VOCAB = 4096
_N = 8 * 2048
_NC, _NS, _L = 2, 16, 16
_NW = _NC * _NS
_ROWS_PER_W = _N // _NW
_K = 8
_NCHUNK = _ROWS_PER_W // _K
_NPAIR = _NCHUNK // 2
_TPAD = _ROWS_PER_W + _L


def _lse_body(w_ref, out_ref):
    x = w_ref[...]
    m = jnp.max(x, axis=1, keepdims=True)
    s = jnp.sum(jnp.exp(x - m), axis=1, keepdims=True)
    out_ref[...] = m + jnp.log(s)


_R = 256


def _compute_lse(W):
    out = pl.pallas_call(
        _lse_body,
        grid=(_VOCAB // _R,),
        in_specs=[pl.BlockSpec((_R, _VOCAB), lambda i: (i, 0))],
        out_specs=pl.BlockSpec((_R, 1), lambda i: (i, 0)),
        out_shape=jax.ShapeDtypeStruct((_VOCAB, 1), jnp.float32),
    )(W)
    return out.reshape(_VOCAB)


@functools.partial(
    pl.kernel,
    mesh=plsc.VectorSubcoreMesh(core_axis_name="c", subcore_axis_name="s"),
    out_type=[
        jax.ShapeDtypeStruct((_N, _VOCAB), jnp.float32),
        jax.ShapeDtypeStruct((_NW, _L), jnp.float32),
    ],
    scratch_types=[
        pltpu.VMEM((_ROWS_PER_W,), jnp.int32),
        pltpu.VMEM((_TPAD,), jnp.int32),
        pltpu.VMEM((_ROWS_PER_W,), jnp.float32),
        pltpu.VMEM((_K, _VOCAB), jnp.float32),
        pltpu.VMEM((_K, _VOCAB), jnp.float32),
        pltpu.VMEM((_L,), jnp.float32),
        pltpu.SemaphoreType.DMA,
        pltpu.SemaphoreType.DMA,
        pltpu.SemaphoreType.DMA,
        pltpu.SemaphoreType.DMA,
        pltpu.SemaphoreType.DMA,
    ],
)
def _sc_gather(W_hbm, idx_hbm, tgt_hbm, lse_hbm, out_hbm, part_hbm,
               idx_v, tgt_v, lv_v, rows_a, rows_b, acc_v,
               gsem_a, gsem_b, wsem_a, wsem_b, lsem):
    wid = lax.axis_index("s") * _NC + lax.axis_index("c")
    base = wid * _ROWS_PER_W
    pltpu.sync_copy(idx_hbm.at[pl.ds(base, _ROWS_PER_W)], idx_v)
    pltpu.sync_copy(tgt_hbm.at[pl.ds(base, _ROWS_PER_W)],
                    tgt_v.at[pl.ds(0, _ROWS_PER_W)])

    cp_l = pltpu.async_copy(lse_hbm.at[idx_v], lv_v, lsem)

    lanes = lax.iota(jnp.int32, _L)

    def _gather(off, buf, sem):
        return pltpu.async_copy(W_hbm.at[idx_v.at[pl.ds(off, _K)]], buf, sem)

    def _gather_wait(off, buf, sem):
        pltpu.make_async_copy(W_hbm.at[idx_v.at[pl.ds(off, _K)]], buf,
                              sem).wait()

    def _wb(off, buf, sem):
        return pltpu.async_copy(buf, out_hbm.at[pl.ds(base + off, _K)], sem)

    def _wb_wait(off, buf, sem):
        pltpu.make_async_copy(buf, out_hbm.at[pl.ds(base + off, _K)],
                              sem).wait()

    def _chunk_tvals(buf, off, accv):
        t16 = tgt_v[pl.ds(off, _L)]
        for j in range(_K):
            t_j = t16[j]
            cbase = (t_j // _L) * _L
            sl = buf[j, pl.ds(cbase, _L)]
            accv = accv + jnp.where(lanes == (t_j % _L), sl, 0.0)
        return accv

    _gather(0, rows_a, gsem_a)

    def body(m, acct):
        e_off = (2 * m) * _K
        o_off = e_off + _K
        _gather(o_off, rows_b, gsem_b)
        _gather_wait(e_off, rows_a, gsem_a)
        _wb(e_off, rows_a, wsem_a)
        acct = _chunk_tvals(rows_a, e_off, acct)
        _gather_wait(o_off, rows_b, gsem_b)
        _wb(o_off, rows_b, wsem_b)
        acct = _chunk_tvals(rows_b, o_off, acct)
        _wb_wait(e_off, rows_a, wsem_a)

        @pl.when(m < _NPAIR - 1)
        def _():
            _gather(e_off + 2 * _K, rows_a, gsem_a)

        _wb_wait(o_off, rows_b, wsem_b)
        return acct

    acct = lax.fori_loop(0, _NPAIR, body, jnp.zeros((_L,), jnp.float32))

    cp_l.wait()

    def lbody(j, accv):
        return accv + lv_v[pl.ds(j * _L, _L)]

    accv = lax.fori_loop(0, _ROWS_PER_W // _L, lbody,
                         jnp.zeros((_L,), jnp.float32))
    acc_v[...] = accv - acct
    pltpu.sync_copy(acc_v, part_hbm.at[wid])


def kernel(idx, targets, W):
    idx_flat = idx.reshape(_N).astype(jnp.int32)
    tgt_flat = targets.reshape(_N).astype(jnp.int32)
    lse = _compute_lse(W)
    logits_flat, partials = _sc_gather(W, idx_flat, tgt_flat, lse)
    loss = jnp.sum(partials) / _N
    return (logits_flat, loss)


# speedup vs baseline: 2.3756x; 1.0146x over previous
import functools

import jax
import jax.numpy as jnp
from jax import lax
from jax.experimental import pallas as pl
from jax.experimental.pallas import tpu as pltpu
from jax.experimental.pallas import tpu_sc as plsc

_VOCAB = 4096
_N = 8 * 2048
_NC, _NS, _L = 2, 16, 16
_NW = _NC * _NS
_ROWS_PER_W = _N // _NW
_K = 4
_NB = 4
_NCHUNK = _ROWS_PER_W // _K
_NG = _NCHUNK // _NB
_TPAD = _ROWS_PER_W + _L


def _lse_body(w_ref, out_ref):
    x = w_ref[...]
    m = jnp.max(x, axis=1, keepdims=True)
    s = jnp.sum(jnp.exp(x - m), axis=1, keepdims=True)
    out_ref[...] = m + jnp.log(s)


_R = 256


def _compute_lse(W):
    return pl.pallas_call(
        _lse_body,
        grid=(_VOCAB // _R,),
        in_specs=[pl.BlockSpec((_R, _VOCAB), lambda i: (i, 0))],
        out_specs=pl.BlockSpec((_R, 1), lambda i: (i, 0)),
        out_shape=jax.ShapeDtypeStruct((_VOCAB, 1), jnp.float32),
    )(W)


def _combine_body(hist_ref, lse_ref, tpart_ref, out_ref):
    counts = jnp.sum(hist_ref[...].astype(jnp.float32), axis=0,
                     keepdims=True)
    s_lse = jnp.dot(counts, lse_ref[...],
                    preferred_element_type=jnp.float32)
    s_tv = jnp.sum(tpart_ref[...])
    out_ref[...] = (s_lse - s_tv) / _N


def _combine(hist, lse2, tpart):
    return pl.pallas_call(
        _combine_body,
        out_shape=jax.ShapeDtypeStruct((1, 1), jnp.float32),
    )(hist, lse2, tpart)


@functools.partial(
    pl.kernel,
    mesh=plsc.VectorSubcoreMesh(core_axis_name="c", subcore_axis_name="s"),
    out_type=[
        jax.ShapeDtypeStruct((_N, _VOCAB), jnp.float32),
        jax.ShapeDtypeStruct((_NW, _L), jnp.float32),
    ],
    scratch_types=[
        pltpu.VMEM((_ROWS_PER_W,), jnp.int32),
        pltpu.VMEM((_NCHUNK, _K), jnp.int32),
        pltpu.VMEM((_ROWS_PER_W,), jnp.int32),
        pltpu.VMEM((_ROWS_PER_W,), jnp.float32),
        pltpu.VMEM((_NB, _K, _VOCAB), jnp.float32),
        pltpu.VMEM((_L,), jnp.float32),
        pltpu.SemaphoreType.DMA((_NB,)),
        pltpu.SemaphoreType.DMA((_NB,)),
        pltpu.SemaphoreType.DMA,
    ],
)
def _sc_gather(W_hbm, idx_hbm, idx2_hbm, tgt_hbm, lse_hbm, out_hbm, part_hbm,
               idx_v, idx2_v, tgt_v, lv_v, rows_v, acc_v, gsems, wsems, lsem):
    wid = lax.axis_index("s") * _NC + lax.axis_index("c")
    base = wid * _ROWS_PER_W
    pltpu.sync_copy(idx_hbm.at[pl.ds(base, _ROWS_PER_W)], idx_v)
    pltpu.sync_copy(idx2_hbm.at[pl.ds(wid * _NCHUNK, _NCHUNK)], idx2_v)
    pltpu.sync_copy(tgt_hbm.at[pl.ds(base, _ROWS_PER_W)], tgt_v)

    lanes = lax.iota(jnp.int32, _L)

    def _gather(c, b):
        return pltpu.async_copy(
            W_hbm.at[idx2_v.at[c]], rows_v.at[b], gsems.at[b])

    def _gather_wait(c, b):
        pltpu.make_async_copy(
            W_hbm.at[idx2_v.at[c]], rows_v.at[b],
            gsems.at[b]).wait()

    def _wb(c, b):
        return pltpu.async_copy(
            rows_v.at[b], out_hbm.at[pl.ds(base + c * _K, _K)], wsems.at[b])

    def _wb_wait(c, b):
        pltpu.make_async_copy(
            rows_v.at[b], out_hbm.at[pl.ds(base + c * _K, _K)],
            wsems.at[b]).wait()

    _gather(0, 0)
    _gather(1, 1)

    cp_l = pltpu.async_copy(lse_hbm.at[idx_v], lv_v, lsem)

    def _chunk_tvals(b, t16, accv):
        for j in range(_K):
            t_j = t16[b * _K + j]
            cbase = (t_j // _L) * _L
            sl = rows_v[b, j, pl.ds(cbase, _L)]
            accv = accv + jnp.where(lanes == (t_j % _L), sl, 0.0)
        return accv

    def body(g, acct):
        t16 = tgt_v[pl.ds(g * _L, _L)]
        for b in range(_NB):
            c = g * _NB + b
            _gather_wait(c, b)
            _wb(c, b)
            acct = _chunk_tvals(b, t16, acct)
            b2 = (b + 2) % _NB
            if b < 2:
                @pl.when(g > 0)
                def _():
                    _wb_wait(c - 2, b2)
                _gather(c + 2, b2)
            else:
                _wb_wait(c - 2, b2)

                @pl.when(g < _NG - 1)
                def _():
                    _gather(c + 2, b2)
        return acct

    acct = lax.fori_loop(0, _NG, body, jnp.zeros((_L,), jnp.float32))
    _wb_wait(_NCHUNK - 2, (_NCHUNK - 2) % _NB)
    _wb_wait(_NCHUNK - 1, (_NCHUNK - 1) % _NB)

    cp_l.wait()

    def lbody(j, accv):
        return accv + lv_v[pl.ds(j * _L, _L)]

    accv = lax.fori_loop(0, _ROWS_PER_W // _L, lbody,
                         jnp.zeros((_L,), jnp.float32))
    acc_v[...] = accv - acct
    pltpu.sync_copy(acc_v, part_hbm.at[wid])


def kernel(idx, targets, W):
    idx_flat = idx.reshape(_N).astype(jnp.int32)
    tgt_flat = targets.reshape(_N).astype(jnp.int32)
    lse = _compute_lse(W).reshape(_VOCAB)
    logits_flat, partials = _sc_gather(
        W, idx_flat, idx_flat.reshape(_N // _K, _K), tgt_flat, lse)
    loss = jnp.sum(partials) / _N
    return (logits_flat, loss)


# speedup vs baseline: 2.5350x; 1.0671x over previous
import functools

import jax
import jax.numpy as jnp
from jax import lax
from jax.experimental import pallas as pl
from jax.experimental.pallas import tpu as pltpu
from jax.experimental.pallas import tpu_sc as plsc

_VOCAB = 4096
_N = 8 * 2048
_NC, _NS, _L = 2, 16, 16
_NW = _NC * _NS
_ROWS_PER_W = _N // _NW
_K = 4
_NB = 4
_NCHUNK = _ROWS_PER_W // _K
_NG = _NCHUNK // _NB
_TPAD = _ROWS_PER_W + _L


def _lse_body(w_ref, out_ref):
    x = w_ref[...]
    m = jnp.max(x, axis=1, keepdims=True)
    s = jnp.sum(jnp.exp(x - m), axis=1, keepdims=True)
    out_ref[...] = m + jnp.log(s)


_R = 256


def _compute_lse(W):
    return pl.pallas_call(
        _lse_body,
        grid=(_VOCAB // _R,),
        in_specs=[pl.BlockSpec((_R, _VOCAB), lambda i: (i, 0))],
        out_specs=pl.BlockSpec((_R, 1), lambda i: (i, 0)),
        out_shape=jax.ShapeDtypeStruct((_VOCAB, 1), jnp.float32),
    )(W)


def _combine_body(hist_ref, lse_ref, tpart_ref, out_ref):
    counts = jnp.sum(hist_ref[...].astype(jnp.float32), axis=0,
                     keepdims=True)
    s_lse = jnp.dot(counts, lse_ref[...],
                    preferred_element_type=jnp.float32)
    s_tv = jnp.sum(tpart_ref[...])
    out_ref[...] = (s_lse - s_tv) / _N


def _combine(hist, lse2, tpart):
    return pl.pallas_call(
        _combine_body,
        out_shape=jax.ShapeDtypeStruct((1, 1), jnp.float32),
    )(hist, lse2, tpart)


@functools.partial(
    pl.kernel,
    mesh=plsc.VectorSubcoreMesh(core_axis_name="c", subcore_axis_name="s"),
    out_type=[
        jax.ShapeDtypeStruct((_N, _VOCAB), jnp.float32),
        jax.ShapeDtypeStruct((_NW, _L), jnp.float32),
        jax.ShapeDtypeStruct((_NC, _VOCAB), jnp.float32),
    ],
    scratch_types=[
        pltpu.VMEM((_ROWS_PER_W,), jnp.int32),
        pltpu.VMEM((_NCHUNK, _K), jnp.int32),
        pltpu.VMEM((_ROWS_PER_W,), jnp.int32),
        pltpu.VMEM((_ROWS_PER_W,), jnp.float32),
        pltpu.VMEM((_VOCAB,), jnp.float32),
        pltpu.VMEM((_NB, _K, _VOCAB), jnp.float32),
        pltpu.VMEM((_L,), jnp.float32),
        pltpu.VMEM_SHARED((_VOCAB,), jnp.float32),
        pltpu.SemaphoreType.DMA((_NB,)),
        pltpu.SemaphoreType.DMA((_NB,)),
    ],
)
def _sc_gather(W_hbm, idx_hbm, idx2_hbm, tgt_hbm, out_hbm, part_hbm, hist_hbm,
               idx_v, idx2_v, tgt_v, ones_v, zsrc_v, rows_v, acc_v, hist_sh,
               gsems, wsems):
    sid = lax.axis_index("s")
    cid = lax.axis_index("c")
    wid = sid * _NC + cid
    base = wid * _ROWS_PER_W
    pltpu.sync_copy(idx_hbm.at[pl.ds(base, _ROWS_PER_W)], idx_v)
    pltpu.sync_copy(idx2_hbm.at[pl.ds(wid * _NCHUNK, _NCHUNK)], idx2_v)
    pltpu.sync_copy(tgt_hbm.at[pl.ds(base, _ROWS_PER_W)], tgt_v)

    lanes = lax.iota(jnp.int32, _L)

    def _gather(c, b):
        return pltpu.async_copy(
            W_hbm.at[idx2_v.at[c]], rows_v.at[b], gsems.at[b])

    def _gather_wait(c, b):
        pltpu.make_async_copy(
            W_hbm.at[idx2_v.at[c]], rows_v.at[b],
            gsems.at[b]).wait()

    def _wb(c, b):
        return pltpu.async_copy(
            rows_v.at[b], out_hbm.at[pl.ds(base + c * _K, _K)], wsems.at[b])

    def _wb_wait(c, b):
        pltpu.make_async_copy(
            rows_v.at[b], out_hbm.at[pl.ds(base + c * _K, _K)],
            wsems.at[b]).wait()

    _gather(0, 0)
    _gather(1, 1)

    def obody(j, _):
        ones_v[pl.ds(j * _L, _L)] = jnp.ones((_L,), jnp.float32)
        return 0

    lax.fori_loop(0, _ROWS_PER_W // _L, obody, 0)

    @pl.when(sid == 0)
    def _():
        def zb(j, _):
            zsrc_v[pl.ds(j * _L, _L)] = jnp.zeros((_L,), jnp.float32)
            return 0
        lax.fori_loop(0, _VOCAB // _L, zb, 0)
        pltpu.sync_copy(zsrc_v, hist_sh)

    plsc.subcore_barrier()
    pltpu.sync_copy(ones_v, hist_sh.at[idx_v], add=True)
    plsc.subcore_barrier()

    @pl.when(sid == 0)
    def _():
        pltpu.sync_copy(hist_sh, hist_hbm.at[cid])

    def _chunk_tvals(b, t16, accv):
        for j in range(_K):
            t_j = t16[b * _K + j]
            cbase = (t_j // _L) * _L
            sl = rows_v[b, j, pl.ds(cbase, _L)]
            accv = accv + jnp.where(lanes == (t_j % _L), sl, 0.0)
        return accv

    def body(g, acct):
        t16 = tgt_v[pl.ds(g * _L, _L)]
        for b in range(_NB):
            c = g * _NB + b
            _gather_wait(c, b)
            _wb(c, b)
            acct = _chunk_tvals(b, t16, acct)
            b2 = (b + 2) % _NB
            if b < 2:
                @pl.when(g > 0)
                def _():
                    _wb_wait(c - 2, b2)
                _gather(c + 2, b2)
            else:
                _wb_wait(c - 2, b2)

                @pl.when(g < _NG - 1)
                def _():
                    _gather(c + 2, b2)
        return acct

    acct = lax.fori_loop(0, _NG, body, jnp.zeros((_L,), jnp.float32))
    _wb_wait(_NCHUNK - 2, (_NCHUNK - 2) % _NB)
    _wb_wait(_NCHUNK - 1, (_NCHUNK - 1) % _NB)

    acc_v[...] = acct
    pltpu.sync_copy(acc_v, part_hbm.at[wid])


def kernel(idx, targets, W):
    idx_flat = idx.reshape(_N).astype(jnp.int32)
    tgt_flat = targets.reshape(_N).astype(jnp.int32)
    lse2 = _compute_lse(W)
    logits_flat, tpart, hist = _sc_gather(
        W, idx_flat, idx_flat.reshape(_N // _K, _K), tgt_flat)
    loss = _combine(hist, lse2, tpart)[0, 0]
    return (logits_flat, loss)
